# baseline (speedup 1.0000x reference)
import functools

import jax
import jax.numpy as jnp
from jax.experimental import pallas as pl
from jax.experimental.pallas import tpu as pltpu


def _uv_body(xt_ref, at_ref, b_ref, u_ref):
    u_ref[0] = (
        jax.lax.dot_general(
            xt_ref[0], at_ref[...], (((1,), (0,)), ((), ())),
            preferred_element_type=jnp.float32,
        )
        + b_ref[...]
    )


def _select_body(k, R, N, xtr_ref, x_ref, xt_ref, u_ref, w2t_ref,
                 ymax_ref, ymin_ref, s1_ref, s2_ref, d_ref, ysum_s, ysq_s):
    b_i = pl.program_id(0)
    rb = pl.program_id(1)
    xb = x_ref[0]
    xtr = xtr_ref[0]
    xt_full = xt_ref[0]
    u_t = u_ref[0]
    w2t = w2t_ref[...]

    sq = jnp.sum(xb * xb, axis=0, keepdims=True)
    g = jax.lax.dot_general(xtr, xb, (((1,), (0,)), ((), ())),
                            preferred_element_type=jnp.float32)
    d_ref[...] = sq - 2.0 * g

    ymax_ref[0] = jnp.full((R, u_t.shape[1]), -jnp.inf, jnp.float32)
    ymin_ref[0] = jnp.full((R, u_t.shape[1]), jnp.inf, jnp.float32)
    ysum_s[...] = jnp.zeros_like(ysum_s)
    ysq_s[...] = jnp.zeros_like(ysq_s)

    thr = jnp.full((R, 1), -jnp.inf, jnp.float32)
    for j in range(k + 1):
        t = d_ref[...]
        w = jnp.where(t > thr, t, jnp.inf)
        m = jnp.min(w, axis=1, keepdims=True)
        msk = w <= m
        thr = m

        if j > 0:
            nbr = jax.lax.dot_general(
                msk.astype(jnp.float32), xt_full, (((1,), (0,)), ((), ())),
                preferred_element_type=jnp.float32)
            y = u_t + jax.lax.dot_general(
                nbr, w2t, (((1,), (0,)), ((), ())),
                preferred_element_type=jnp.float32)
            ymax_ref[0] = jnp.maximum(ymax_ref[0], y)
            ymin_ref[0] = jnp.minimum(ymin_ref[0], y)
            ysum_s[...] = ysum_s[...] + y
            ysq_s[...] = ysq_s[...] + y * y

    @pl.when(jnp.logical_and(b_i == 0, rb == 0))
    def _():
        s1_ref[...] = jnp.zeros_like(s1_ref)
        s2_ref[...] = jnp.zeros_like(s2_ref)

    s1_ref[...] = s1_ref[...] + jnp.sum(ysum_s[...], axis=0, keepdims=True)
    s2_ref[...] = s2_ref[...] + jnp.sum(ysq_s[...], axis=0, keepdims=True)


def _final_body(cnt, ymax_ref, ymin_ref, s1_ref, s2_ref, g_ref, be_ref,
                out_ref):
    mean = s1_ref[...] / cnt
    var = s2_ref[...] / cnt - mean * mean
    scale = g_ref[...] * jax.lax.rsqrt(var + 1e-5)
    mv = jnp.where(scale >= 0.0, ymax_ref[0], ymin_ref[0])
    out_ref[0] = jnp.maximum((mv - mean) * scale + be_ref[...], 0.0)


def kernel(x, W, b, gamma, beta):
    B, C, N = x.shape
    O = W.shape[0]
    k = min(20, N - 1)
    R = min(256, N)
    NB = N // R
    cnt = float(B * N * k)

    xt = jnp.transpose(x, (0, 2, 1))
    at = jnp.transpose(W[:, :C] - W[:, C:])
    w2t = jnp.transpose(W[:, C:])
    b2 = b.reshape(1, O)
    g2 = gamma.reshape(1, O)
    be2 = beta.reshape(1, O)

    u = pl.pallas_call(
        _uv_body,
        grid=(B,),
        in_specs=[
            pl.BlockSpec((1, N, C), lambda i: (i, 0, 0)),
            pl.BlockSpec((C, O), lambda i: (0, 0)),
            pl.BlockSpec((1, O), lambda i: (0, 0)),
        ],
        out_specs=pl.BlockSpec((1, N, O), lambda i: (i, 0, 0)),
        out_shape=jax.ShapeDtypeStruct((B, N, O), jnp.float32),
    )(xt, at, b2)

    ymax, ymin, s1, s2 = pl.pallas_call(
        functools.partial(_select_body, k, R, N),
        grid=(B, NB),
        in_specs=[
            pl.BlockSpec((1, R, C), lambda bi, ri: (bi, ri, 0)),
            pl.BlockSpec((1, C, N), lambda bi, ri: (bi, 0, 0)),
            pl.BlockSpec((1, N, C), lambda bi, ri: (bi, 0, 0)),
            pl.BlockSpec((1, R, O), lambda bi, ri: (bi, ri, 0)),
            pl.BlockSpec((C, O), lambda bi, ri: (0, 0)),
        ],
        out_specs=[
            pl.BlockSpec((1, R, O), lambda bi, ri: (bi, ri, 0)),
            pl.BlockSpec((1, R, O), lambda bi, ri: (bi, ri, 0)),
            pl.BlockSpec((1, O), lambda bi, ri: (0, 0)),
            pl.BlockSpec((1, O), lambda bi, ri: (0, 0)),
        ],
        out_shape=[
            jax.ShapeDtypeStruct((B, N, O), jnp.float32),
            jax.ShapeDtypeStruct((B, N, O), jnp.float32),
            jax.ShapeDtypeStruct((1, O), jnp.float32),
            jax.ShapeDtypeStruct((1, O), jnp.float32),
        ],
        scratch_shapes=[
            pltpu.VMEM((R, N), jnp.float32),
            pltpu.VMEM((R, O), jnp.float32),
            pltpu.VMEM((R, O), jnp.float32),
        ],
    )(xt, x, xt, u, w2t)

    out_bno = pl.pallas_call(
        functools.partial(_final_body, cnt),
        grid=(B, NB),
        in_specs=[
            pl.BlockSpec((1, R, O), lambda bi, ri: (bi, ri, 0)),
            pl.BlockSpec((1, R, O), lambda bi, ri: (bi, ri, 0)),
            pl.BlockSpec((1, O), lambda bi, ri: (0, 0)),
            pl.BlockSpec((1, O), lambda bi, ri: (0, 0)),
            pl.BlockSpec((1, O), lambda bi, ri: (0, 0)),
            pl.BlockSpec((1, O), lambda bi, ri: (0, 0)),
        ],
        out_specs=pl.BlockSpec((1, R, O), lambda bi, ri: (bi, ri, 0)),
        out_shape=jax.ShapeDtypeStruct((B, N, O), jnp.float32),
    )(ymax, ymin, s1, s2, g2, be2)

    return jnp.transpose(out_bno, (0, 2, 1))


# speedup vs baseline: 1.0050x; 1.0050x over previous
import functools

import jax
import jax.numpy as jnp
from jax.experimental import pallas as pl
from jax.experimental.pallas import tpu as pltpu


def _uv_body(xt_ref, at_ref, b_ref, u_ref):
    u_ref[0] = (
        jax.lax.dot_general(
            xt_ref[0], at_ref[...], (((1,), (0,)), ((), ())),
            preferred_element_type=jnp.float32,
        )
        + b_ref[...]
    )


def _select_body(k, R, N, xtr_ref, x_ref, xt_ref, u_ref, w2t_ref,
                 ymax_ref, ymin_ref, s1_ref, s2_ref, d_ref):
    b_i = pl.program_id(0)
    rb = pl.program_id(1)
    xb = x_ref[0]
    xtr = xtr_ref[0]
    xt_full = xt_ref[0]
    u_t = u_ref[0]
    w2t = w2t_ref[...]

    sq = jnp.sum(xb * xb, axis=0, keepdims=True)
    g = jax.lax.dot_general(xtr, xb, (((1,), (0,)), ((), ())),
                            preferred_element_type=jnp.float32)
    d_ref[...] = sq - 2.0 * g

    O = u_t.shape[1]
    ymax_v = jnp.full((R, O), -jnp.inf, jnp.float32)
    ymin_v = jnp.full((R, O), jnp.inf, jnp.float32)
    ysum_v = jnp.zeros((R, O), jnp.float32)
    ysq_v = jnp.zeros((R, O), jnp.float32)

    thr = jnp.full((R, 1), -jnp.inf, jnp.float32)
    for j in range(k + 1):
        t = d_ref[...]
        w = jnp.where(t > thr, t, jnp.inf)
        m = jnp.min(w, axis=1, keepdims=True)
        msk = t == m
        thr = m

        if j > 0:
            nbr = jax.lax.dot_general(
                msk.astype(jnp.float32), xt_full, (((1,), (0,)), ((), ())),
                preferred_element_type=jnp.float32)
            y = u_t + jax.lax.dot_general(
                nbr, w2t, (((1,), (0,)), ((), ())),
                preferred_element_type=jnp.float32)
            ymax_v = jnp.maximum(ymax_v, y)
            ymin_v = jnp.minimum(ymin_v, y)
            ysum_v = ysum_v + y
            ysq_v = ysq_v + y * y

    ymax_ref[0] = ymax_v
    ymin_ref[0] = ymin_v

    @pl.when(jnp.logical_and(b_i == 0, rb == 0))
    def _():
        s1_ref[...] = jnp.zeros_like(s1_ref)
        s2_ref[...] = jnp.zeros_like(s2_ref)

    s1_ref[...] = s1_ref[...] + jnp.sum(ysum_v, axis=0, keepdims=True)
    s2_ref[...] = s2_ref[...] + jnp.sum(ysq_v, axis=0, keepdims=True)


def _final_body(cnt, ymax_ref, ymin_ref, s1_ref, s2_ref, g_ref, be_ref,
                out_ref):
    mean = s1_ref[...] / cnt
    var = s2_ref[...] / cnt - mean * mean
    scale = g_ref[...] * jax.lax.rsqrt(var + 1e-5)
    mv = jnp.where(scale >= 0.0, ymax_ref[0], ymin_ref[0])
    out_ref[0] = jnp.maximum((mv - mean) * scale + be_ref[...], 0.0)


def kernel(x, W, b, gamma, beta):
    B, C, N = x.shape
    O = W.shape[0]
    k = min(20, N - 1)
    R = min(256, N)
    NB = N // R
    cnt = float(B * N * k)

    xt = jnp.transpose(x, (0, 2, 1))
    at = jnp.transpose(W[:, :C] - W[:, C:])
    w2t = jnp.transpose(W[:, C:])
    b2 = b.reshape(1, O)
    g2 = gamma.reshape(1, O)
    be2 = beta.reshape(1, O)

    u = pl.pallas_call(
        _uv_body,
        grid=(B,),
        in_specs=[
            pl.BlockSpec((1, N, C), lambda i: (i, 0, 0)),
            pl.BlockSpec((C, O), lambda i: (0, 0)),
            pl.BlockSpec((1, O), lambda i: (0, 0)),
        ],
        out_specs=pl.BlockSpec((1, N, O), lambda i: (i, 0, 0)),
        out_shape=jax.ShapeDtypeStruct((B, N, O), jnp.float32),
    )(xt, at, b2)

    ymax, ymin, s1, s2 = pl.pallas_call(
        functools.partial(_select_body, k, R, N),
        grid=(B, NB),
        in_specs=[
            pl.BlockSpec((1, R, C), lambda bi, ri: (bi, ri, 0)),
            pl.BlockSpec((1, C, N), lambda bi, ri: (bi, 0, 0)),
            pl.BlockSpec((1, N, C), lambda bi, ri: (bi, 0, 0)),
            pl.BlockSpec((1, R, O), lambda bi, ri: (bi, ri, 0)),
            pl.BlockSpec((C, O), lambda bi, ri: (0, 0)),
        ],
        out_specs=[
            pl.BlockSpec((1, R, O), lambda bi, ri: (bi, ri, 0)),
            pl.BlockSpec((1, R, O), lambda bi, ri: (bi, ri, 0)),
            pl.BlockSpec((1, O), lambda bi, ri: (0, 0)),
            pl.BlockSpec((1, O), lambda bi, ri: (0, 0)),
        ],
        out_shape=[
            jax.ShapeDtypeStruct((B, N, O), jnp.float32),
            jax.ShapeDtypeStruct((B, N, O), jnp.float32),
            jax.ShapeDtypeStruct((1, O), jnp.float32),
            jax.ShapeDtypeStruct((1, O), jnp.float32),
        ],
        scratch_shapes=[
            pltpu.VMEM((R, N), jnp.float32),
        ],
    )(xt, x, xt, u, w2t)

    out_bno = pl.pallas_call(
        functools.partial(_final_body, cnt),
        grid=(B, NB),
        in_specs=[
            pl.BlockSpec((1, R, O), lambda bi, ri: (bi, ri, 0)),
            pl.BlockSpec((1, R, O), lambda bi, ri: (bi, ri, 0)),
            pl.BlockSpec((1, O), lambda bi, ri: (0, 0)),
            pl.BlockSpec((1, O), lambda bi, ri: (0, 0)),
            pl.BlockSpec((1, O), lambda bi, ri: (0, 0)),
            pl.BlockSpec((1, O), lambda bi, ri: (0, 0)),
        ],
        out_specs=pl.BlockSpec((1, R, O), lambda bi, ri: (bi, ri, 0)),
        out_shape=jax.ShapeDtypeStruct((B, N, O), jnp.float32),
    )(ymax, ymin, s1, s2, g2, be2)

    return jnp.transpose(out_bno, (0, 2, 1))


# speedup vs baseline: 1.0521x; 1.0468x over previous
import functools

import jax
import jax.numpy as jnp
from jax.experimental import pallas as pl
from jax.experimental.pallas import tpu as pltpu


def _uv_body(xt_ref, at_ref, b_ref, u_ref):
    u_ref[0] = (
        jax.lax.dot_general(
            xt_ref[0], at_ref[...], (((1,), (0,)), ((), ())),
            preferred_element_type=jnp.float32,
        )
        + b_ref[...]
    )


def _select_body(k, R, N, xtr_ref, x_ref, xt_ref, u_ref, w2t_ref,
                 ymax_ref, ymin_ref, s1_ref, s2_ref, d_ref):
    b_i = pl.program_id(0)
    rb = pl.program_id(1)
    xb = x_ref[0]
    xtr = xtr_ref[0]
    xt_full = xt_ref[0]
    u_t = u_ref[0]
    w2t = w2t_ref[...]

    sq = jnp.sum(xb * xb, axis=0, keepdims=True)
    g = jax.lax.dot_general(xtr, xb, (((1,), (0,)), ((), ())),
                            preferred_element_type=jnp.float32)
    d_ref[...] = sq - 2.0 * g

    O = u_t.shape[1]
    ymax_v = jnp.full((R, O), -jnp.inf, jnp.float32)
    ymin_v = jnp.full((R, O), jnp.inf, jnp.float32)
    ysum_v = jnp.zeros((R, O), jnp.float32)
    ysq_v = jnp.zeros((R, O), jnp.float32)

    thr = jnp.full((R, 1), -jnp.inf, jnp.float32)
    for j in range(k + 1):
        t = d_ref[...]
        w = jnp.where(t > thr, t, jnp.inf)
        m = jnp.min(w, axis=1, keepdims=True)
        msk = t == m
        thr = m

        if j > 0:
            nbr = jax.lax.dot_general(
                msk.astype(jnp.float32), xt_full, (((1,), (0,)), ((), ())),
                preferred_element_type=jnp.float32)
            y = u_t + jax.lax.dot_general(
                nbr, w2t, (((1,), (0,)), ((), ())),
                preferred_element_type=jnp.float32)
            ymax_v = jnp.maximum(ymax_v, y)
            ymin_v = jnp.minimum(ymin_v, y)
            ysum_v = ysum_v + y
            ysq_v = ysq_v + y * y

    ymax_ref[0] = ymax_v
    ymin_ref[0] = ymin_v

    @pl.when(jnp.logical_and(b_i == 0, rb == 0))
    def _():
        s1_ref[...] = jnp.zeros_like(s1_ref)
        s2_ref[...] = jnp.zeros_like(s2_ref)

    s1_ref[...] = s1_ref[...] + jnp.sum(ysum_v, axis=0, keepdims=True)
    s2_ref[...] = s2_ref[...] + jnp.sum(ysq_v, axis=0, keepdims=True)


def _final_body(cnt, ymax_ref, ymin_ref, s1_ref, s2_ref, g_ref, be_ref,
                out_ref):
    mean = s1_ref[...] / cnt
    var = s2_ref[...] / cnt - mean * mean
    scale = g_ref[...] * jax.lax.rsqrt(var + 1e-5)
    mv = jnp.where(scale >= 0.0, ymax_ref[0], ymin_ref[0])
    out_ref[0] = jnp.maximum((mv - mean) * scale + be_ref[...], 0.0)


def kernel(x, W, b, gamma, beta):
    B, C, N = x.shape
    O = W.shape[0]
    k = min(20, N - 1)
    R = min(512, N)
    NB = N // R
    cnt = float(B * N * k)

    xt = jnp.transpose(x, (0, 2, 1))
    at = jnp.transpose(W[:, :C] - W[:, C:])
    w2t = jnp.transpose(W[:, C:])
    b2 = b.reshape(1, O)
    g2 = gamma.reshape(1, O)
    be2 = beta.reshape(1, O)

    u = pl.pallas_call(
        _uv_body,
        grid=(B,),
        in_specs=[
            pl.BlockSpec((1, N, C), lambda i: (i, 0, 0)),
            pl.BlockSpec((C, O), lambda i: (0, 0)),
            pl.BlockSpec((1, O), lambda i: (0, 0)),
        ],
        out_specs=pl.BlockSpec((1, N, O), lambda i: (i, 0, 0)),
        out_shape=jax.ShapeDtypeStruct((B, N, O), jnp.float32),
    )(xt, at, b2)

    ymax, ymin, s1, s2 = pl.pallas_call(
        functools.partial(_select_body, k, R, N),
        grid=(B, NB),
        in_specs=[
            pl.BlockSpec((1, R, C), lambda bi, ri: (bi, ri, 0)),
            pl.BlockSpec((1, C, N), lambda bi, ri: (bi, 0, 0)),
            pl.BlockSpec((1, N, C), lambda bi, ri: (bi, 0, 0)),
            pl.BlockSpec((1, R, O), lambda bi, ri: (bi, ri, 0)),
            pl.BlockSpec((C, O), lambda bi, ri: (0, 0)),
        ],
        out_specs=[
            pl.BlockSpec((1, R, O), lambda bi, ri: (bi, ri, 0)),
            pl.BlockSpec((1, R, O), lambda bi, ri: (bi, ri, 0)),
            pl.BlockSpec((1, O), lambda bi, ri: (0, 0)),
            pl.BlockSpec((1, O), lambda bi, ri: (0, 0)),
        ],
        out_shape=[
            jax.ShapeDtypeStruct((B, N, O), jnp.float32),
            jax.ShapeDtypeStruct((B, N, O), jnp.float32),
            jax.ShapeDtypeStruct((1, O), jnp.float32),
            jax.ShapeDtypeStruct((1, O), jnp.float32),
        ],
        scratch_shapes=[
            pltpu.VMEM((R, N), jnp.float32),
        ],
    )(xt, x, xt, u, w2t)

    out_bno = pl.pallas_call(
        functools.partial(_final_body, cnt),
        grid=(B, NB),
        in_specs=[
            pl.BlockSpec((1, R, O), lambda bi, ri: (bi, ri, 0)),
            pl.BlockSpec((1, R, O), lambda bi, ri: (bi, ri, 0)),
            pl.BlockSpec((1, O), lambda bi, ri: (0, 0)),
            pl.BlockSpec((1, O), lambda bi, ri: (0, 0)),
            pl.BlockSpec((1, O), lambda bi, ri: (0, 0)),
            pl.BlockSpec((1, O), lambda bi, ri: (0, 0)),
        ],
        out_specs=pl.BlockSpec((1, R, O), lambda bi, ri: (bi, ri, 0)),
        out_shape=jax.ShapeDtypeStruct((B, N, O), jnp.float32),
    )(ymax, ymin, s1, s2, g2, be2)

    return jnp.transpose(out_bno, (0, 2, 1))
